# baseline (speedup 1.0000x reference)
import functools

import jax
import jax.numpy as jnp
from jax import lax
from jax.experimental import pallas as pl
from jax.experimental.pallas import tpu as pltpu
from jax.experimental.pallas import tpu_sc as plsc

_CR = 32
_VLEN = 16


def kernel(x, pe):
    B, S, E = x.shape
    info = plsc.get_sparse_core_info()
    NC, NS = info.num_cores, info.num_subcores
    NW = NC * NS
    rows = B * S
    rpw = rows // NW
    nchunk = rpw // _CR
    chunk_elems = _CR * E
    nvec = chunk_elems // _VLEN

    xf = x.reshape(rows * E)
    pef = pe.reshape(pe.shape[0] * E)
    mesh = plsc.VectorSubcoreMesh(core_axis_name="c", subcore_axis_name="s")

    @functools.partial(
        pl.kernel,
        out_type=jax.ShapeDtypeStruct((rows * E,), x.dtype),
        mesh=mesh,
        scratch_types=[
            pltpu.VMEM((chunk_elems,), jnp.float32),
            pltpu.VMEM((chunk_elems,), jnp.float32),
        ],
    )
    def k(x_hbm, pe_hbm, o_hbm, xbuf, pebuf):
        wid = lax.axis_index("s") * NC + lax.axis_index("c")
        base = wid * rpw

        @pl.loop(0, nchunk)
        def _chunk(ci):
            row0 = base + ci * _CR
            s0 = lax.rem(row0, S)
            pltpu.sync_copy(x_hbm.at[pl.ds(row0 * E, chunk_elems)], xbuf)
            pltpu.sync_copy(pe_hbm.at[pl.ds(s0 * E, chunk_elems)], pebuf)

            @pl.loop(0, nvec)
            def _vec(j):
                off = j * _VLEN
                xbuf[pl.ds(off, _VLEN)] = (
                    xbuf[pl.ds(off, _VLEN)] + pebuf[pl.ds(off, _VLEN)]
                )

            pltpu.sync_copy(xbuf, o_hbm.at[pl.ds(row0 * E, chunk_elems)])

    out = k(xf, pef)
    return out.reshape(B, S, E)


# speedup vs baseline: 8.6264x; 8.6264x over previous
import jax
import jax.numpy as jnp
from jax.experimental import pallas as pl

_S_BLK = 2048


def _body(x_ref, pe_ref, o_ref):
    o_ref[...] = x_ref[...] + pe_ref[...]


def kernel(x, pe):
    B, S, E = x.shape
    grid = (S // _S_BLK, B)
    return pl.pallas_call(
        _body,
        grid=grid,
        in_specs=[
            pl.BlockSpec((1, _S_BLK, E), lambda i, b: (b, i, 0)),
            pl.BlockSpec((_S_BLK, E), lambda i, b: (i, 0)),
        ],
        out_specs=pl.BlockSpec((1, _S_BLK, E), lambda i, b: (b, i, 0)),
        out_shape=jax.ShapeDtypeStruct(x.shape, x.dtype),
    )(x, pe)
